# baseline (speedup 1.0000x reference)
import functools

import jax
import jax.numpy as jnp
from jax import lax
from jax.experimental import pallas as pl
from jax.experimental.pallas import tpu as pltpu
from jax.experimental.pallas import tpu_sc as plsc

N_DRUGS = 1000
N_LINES = 100000
BATCH = 16384
N_CONC = 20

NC = 2
NS = 16
NW = NC * NS
BPW = BATCH // NW
CHUNK = 128
NCH = BPW // CHUNK
LANES = 16
LINE_WORDS = BPW * CHUNK

_mesh = plsc.VectorSubcoreMesh(
    core_axis_name="c", subcore_axis_name="s", num_cores=NC, num_subcores=NS
)

_vec = jax.ShapeDtypeStruct((BATCH,), jnp.float32)


@functools.partial(
    pl.kernel,
    out_type=(_vec, _vec, _vec, _vec),
    mesh=_mesh,
    compiler_params=pltpu.CompilerParams(use_tc_tiling_on_sc=True),
    scratch_types=[
        pltpu.VMEM((BPW,), jnp.int32),
        pltpu.VMEM((BPW,), jnp.int32),
        pltpu.VMEM((NCH, CHUNK), jnp.int32),
        pltpu.VMEM_SHARED((NS * LINE_WORDS,), jnp.float32),
        pltpu.VMEM((BPW,), jnp.float32),
        pltpu.VMEM((BPW,), jnp.float32),
        pltpu.VMEM((BPW,), jnp.float32),
        pltpu.VMEM((BPW,), jnp.float32),
        pltpu.VMEM((BPW,), jnp.float32),
        pltpu.SemaphoreType.DMA,
        pltpu.SemaphoreType.DMA,
    ],
)
def _sc_gather(d_hbm, l_hbm, b1_hbm, b2_hbm, b3_hbm, b4_hbm, b4ld_hbm,
               b1_out, b2_out, b3_out, b4_out,
               lidx, didx, fidx, lines, b1v, b2v, b3v, b4v, b4e, sem, wsem):
    wid = lax.axis_index("s") * NC + lax.axis_index("c")
    base = wid * BPW
    lane_iota = lax.iota(jnp.int32, LANES)
    sid = lax.axis_index("s")
    sbase = sid * LINE_WORDS

    pltpu.sync_copy(l_hbm.at[pl.ds(base, BPW)], lidx)
    pltpu.sync_copy(d_hbm.at[pl.ds(base, BPW)], didx)

    for k in range(NCH):
        for j in range(CHUNK // LANES):
            i0 = k * CHUNK + j * LANES
            sl = pl.ds(i0, LANES)
            dv = didx[sl]
            fidx[k, pl.ds(j * LANES, LANES)] = (
                sbase + (lane_iota + i0) * CHUNK + (dv & (CHUNK - 1)))

    copies = []
    for k in range(NCH):
        isl = pl.ds(k * CHUNK, CHUNK)
        copies.append(pltpu.async_copy(b1_hbm.at[lidx.at[isl]], b1v.at[isl], sem))
        copies.append(pltpu.async_copy(b2_hbm.at[lidx.at[isl]], b2v.at[isl], sem))
        copies.append(pltpu.async_copy(b3_hbm.at[lidx.at[isl]], b3v.at[isl], sem))
        copies.append(pltpu.async_copy(b4_hbm.at[lidx.at[isl]], b4v.at[isl], sem))

    def wave(q, carry):
        lvec = lidx[pl.ds(q * LANES, LANES)]
        dvec = didx[pl.ds(q * LANES, LANES)]
        for u in range(LANES):
            i = q * LANES + u
            dt = (dvec[u] // CHUNK) * CHUNK
            pltpu.async_copy(
                b4ld_hbm.at[pl.ds(lvec[u], 1), pl.ds(dt, CHUNK)].at[0],
                lines.at[pl.ds(sbase + i * CHUNK, CHUNK)], wsem)
        return carry

    lax.fori_loop(0, BPW // LANES, wave, 0)

    pltpu.make_async_copy(
        b1_hbm.at[pl.ds(0, LINE_WORDS)],
        lines.at[pl.ds(sbase, LINE_WORDS)], wsem).wait()

    ecopies = []
    for k in range(NCH):
        dsl = pl.ds(k * CHUNK, CHUNK)
        ecopies.append(pltpu.async_copy(
            lines.at[fidx.at[k]], b4e.at[dsl], wsem))
    for cp in ecopies:
        cp.wait()
    for cp in copies:
        cp.wait()

    for g in range(BPW // LANES):
        sl = pl.ds(g * LANES, LANES)
        b4v[sl] = b4v[sl] + b4e[sl]

    pltpu.sync_copy(b1v, b1_out.at[pl.ds(base, BPW)])
    pltpu.sync_copy(b2v, b2_out.at[pl.ds(base, BPW)])
    pltpu.sync_copy(b3v, b3_out.at[pl.ds(base, BPW)])
    pltpu.sync_copy(b4v, b4_out.at[pl.ds(base, BPW)])


def _tc_body(c_ref, b1_ref, b2_ref, b3_ref, b4_ref, o_ref):
    b1 = b1_ref[...][:, None]
    b2 = b2_ref[...][:, None]
    b3 = b3_ref[...][:, None]
    b4 = b4_ref[...][:, None]
    c = c_ref[...]
    o_ref[...] = b1 + (b2 - b1) * jax.nn.sigmoid(b4 * (c - b3))


_tc_sigmoid = pl.pallas_call(
    _tc_body,
    out_shape=jax.ShapeDtypeStruct((BATCH, N_CONC), jnp.float32),
    grid=(8,),
    in_specs=[
        pl.BlockSpec((BATCH // 8, N_CONC), lambda i: (i, 0)),
        pl.BlockSpec((BATCH // 8,), lambda i: (i,)),
        pl.BlockSpec((BATCH // 8,), lambda i: (i,)),
        pl.BlockSpec((BATCH // 8,), lambda i: (i,)),
        pl.BlockSpec((BATCH // 8,), lambda i: (i,)),
    ],
    out_specs=pl.BlockSpec((BATCH // 8, N_CONC), lambda i: (i, 0)),
)


def kernel(d, l, c, B1l, B2l, B3l, B4l, B4ld):
    b1, b2, b3, b4 = _sc_gather(d, l, B1l, B2l, B3l, B4l, B4ld)
    return _tc_sigmoid(c, b1, b2, b3, b4)


# speedup vs baseline: 7.0366x; 7.0366x over previous
import functools

import jax
import jax.numpy as jnp
from jax import lax
from jax.experimental import pallas as pl
from jax.experimental.pallas import tpu as pltpu
from jax.experimental.pallas import tpu_sc as plsc

N_DRUGS = 1000
N_LINES = 100000
BATCH = 16384
N_CONC = 20

NC = 2
NS = 16
NW = NC * NS
BPW = BATCH // NW
CHUNK = 128
NCH = BPW // CHUNK
LANES = 16
WIN = CHUNK
LINE_WORDS = BPW * WIN

_mesh = plsc.VectorSubcoreMesh(
    core_axis_name="c", subcore_axis_name="s", num_cores=NC, num_subcores=NS
)

_vec = jax.ShapeDtypeStruct((BATCH,), jnp.float32)


@functools.partial(
    pl.kernel,
    out_type=(_vec, _vec, _vec, _vec),
    mesh=_mesh,
    compiler_params=pltpu.CompilerParams(
        use_tc_tiling_on_sc=True,
        disable_bounds_checks=True,
        disable_semaphore_checks=True,
        skip_device_barrier=True,
    ),
    scratch_types=[
        pltpu.VMEM((BPW,), jnp.int32),
        pltpu.VMEM((BPW,), jnp.int32),
        pltpu.VMEM((NCH, CHUNK), jnp.int32),
        pltpu.VMEM_SHARED((NS * LINE_WORDS,), jnp.float32),
        pltpu.VMEM((BPW,), jnp.float32),
        pltpu.VMEM((BPW,), jnp.float32),
        pltpu.VMEM((BPW,), jnp.float32),
        pltpu.VMEM((BPW,), jnp.float32),
        pltpu.VMEM((BPW,), jnp.float32),
        pltpu.SemaphoreType.DMA,
        pltpu.SemaphoreType.DMA,
    ],
)
def _sc_gather(d_hbm, l_hbm, b1_hbm, b2_hbm, b3_hbm, b4_hbm, b4ld_hbm,
               b1_out, b2_out, b3_out, b4_out,
               lidx, didx, fidx, lines, b1v, b2v, b3v, b4v, b4e, sem, wsem):
    wid = lax.axis_index("s") * NC + lax.axis_index("c")
    base = wid * BPW
    lane_iota = lax.iota(jnp.int32, LANES)
    sid = lax.axis_index("s")
    sbase = sid * LINE_WORDS

    pltpu.sync_copy(l_hbm.at[pl.ds(base, BPW)], lidx)
    pltpu.sync_copy(d_hbm.at[pl.ds(base, BPW)], didx)

    def wave(q, carry):
        lvec = lidx[pl.ds(q * LANES, LANES)]
        dvec = didx[pl.ds(q * LANES, LANES)]
        for u in range(LANES):
            i = q * LANES + u
            lt = (lvec[u] // WIN) * WIN
            pltpu.async_copy(
                b4ld_hbm.at[pl.ds(dvec[u], 1), pl.ds(lt, WIN)].at[0],
                lines.at[pl.ds(sbase + i * WIN, WIN)], wsem)
        return carry

    lax.fori_loop(0, BPW // LANES, wave, 0)

    copies = []
    for k in range(NCH):
        isl = pl.ds(k * CHUNK, CHUNK)
        copies.append(pltpu.async_copy(b1_hbm.at[lidx.at[isl]], b1v.at[isl], sem))
        copies.append(pltpu.async_copy(b2_hbm.at[lidx.at[isl]], b2v.at[isl], sem))
        copies.append(pltpu.async_copy(b3_hbm.at[lidx.at[isl]], b3v.at[isl], sem))
        copies.append(pltpu.async_copy(b4_hbm.at[lidx.at[isl]], b4v.at[isl], sem))

    for k in range(NCH):
        for j in range(CHUNK // LANES):
            i0 = k * CHUNK + j * LANES
            sl = pl.ds(i0, LANES)
            lv = lidx[sl]
            fidx[k, pl.ds(j * LANES, LANES)] = (
                sbase + (lane_iota + i0) * WIN + (lv & (WIN - 1)))

    pltpu.make_async_copy(
        b1_hbm.at[pl.ds(0, LINE_WORDS)],
        lines.at[pl.ds(sbase, LINE_WORDS)], wsem).wait()

    ecopies = []
    for k in range(NCH):
        dsl = pl.ds(k * CHUNK, CHUNK)
        ecopies.append(pltpu.async_copy(
            lines.at[fidx.at[k]], b4e.at[dsl], wsem))
    for cp in ecopies:
        cp.wait()
    for cp in copies:
        cp.wait()

    for g in range(BPW // LANES):
        sl = pl.ds(g * LANES, LANES)
        b4v[sl] = b4v[sl] + b4e[sl]

    pltpu.sync_copy(b1v, b1_out.at[pl.ds(base, BPW)])
    pltpu.sync_copy(b2v, b2_out.at[pl.ds(base, BPW)])
    pltpu.sync_copy(b3v, b3_out.at[pl.ds(base, BPW)])
    pltpu.sync_copy(b4v, b4_out.at[pl.ds(base, BPW)])


def _tc_body(ct_ref, b1_ref, b2_ref, b3_ref, b4_ref, o_ref):
    b1 = b1_ref[...][None, :]
    b2 = b2_ref[...][None, :]
    b3 = b3_ref[...][None, :]
    b4 = b4_ref[...][None, :]
    ct = ct_ref[...]
    o_ref[...] = b1 + (b2 - b1) * jax.nn.sigmoid(b4 * (ct - b3))


_tc_sigmoid = pl.pallas_call(
    _tc_body,
    out_shape=jax.ShapeDtypeStruct((N_CONC, BATCH), jnp.float32),
    grid=(4,),
    in_specs=[
        pl.BlockSpec((N_CONC, BATCH // 4), lambda i: (0, i)),
        pl.BlockSpec((BATCH // 4,), lambda i: (i,)),
        pl.BlockSpec((BATCH // 4,), lambda i: (i,)),
        pl.BlockSpec((BATCH // 4,), lambda i: (i,)),
        pl.BlockSpec((BATCH // 4,), lambda i: (i,)),
    ],
    out_specs=pl.BlockSpec((N_CONC, BATCH // 4), lambda i: (0, i)),
)


def kernel(d, l, c, B1l, B2l, B3l, B4l, B4ld):
    b1, b2, b3, b4 = _sc_gather(d, l, B1l, B2l, B3l, B4l, B4ld.T)
    return _tc_sigmoid(c.T, b1, b2, b3, b4).T


# speedup vs baseline: 7.0714x; 1.0050x over previous
import functools

import jax
import jax.numpy as jnp
from jax import lax
from jax.experimental import pallas as pl
from jax.experimental.pallas import tpu as pltpu
from jax.experimental.pallas import tpu_sc as plsc

N_DRUGS = 1000
N_LINES = 100000
BATCH = 16384
N_CONC = 20

NC = 2
NS = 16
NW = NC * NS
BPW = BATCH // NW
CHUNK = 128
NCH = BPW // CHUNK
LANES = 16
WIN = CHUNK
LINE_WORDS = BPW * WIN

_mesh = plsc.VectorSubcoreMesh(
    core_axis_name="c", subcore_axis_name="s", num_cores=NC, num_subcores=NS
)

_vec = jax.ShapeDtypeStruct((BATCH,), jnp.float32)


@functools.partial(
    pl.kernel,
    out_type=(_vec, _vec, _vec, _vec),
    mesh=_mesh,
    compiler_params=pltpu.CompilerParams(
        use_tc_tiling_on_sc=True,
        disable_bounds_checks=True,
        disable_semaphore_checks=True,
        skip_device_barrier=True,
    ),
    scratch_types=[
        pltpu.VMEM((BPW,), jnp.int32),
        pltpu.VMEM((BPW,), jnp.int32),
        pltpu.VMEM((NCH, CHUNK), jnp.int32),
        pltpu.VMEM_SHARED((NS * LINE_WORDS,), jnp.float32),
        pltpu.VMEM((BPW,), jnp.float32),
        pltpu.VMEM((BPW,), jnp.float32),
        pltpu.VMEM((BPW,), jnp.float32),
        pltpu.VMEM((BPW,), jnp.float32),
        pltpu.VMEM((BPW,), jnp.float32),
        pltpu.SemaphoreType.DMA,
        pltpu.SemaphoreType.DMA,
    ],
)
def _sc_gather(d_hbm, l_hbm, b1_hbm, b2_hbm, b3_hbm, b4_hbm, b4ld_hbm,
               b1_out, b2_out, b3_out, b4_out,
               lidx, didx, fidx, lines, b1v, b2v, b3v, b4v, b4e, sem, wsem):
    wid = lax.axis_index("s") * NC + lax.axis_index("c")
    base = wid * BPW
    lane_iota = lax.iota(jnp.int32, LANES)
    sid = lax.axis_index("s")
    sbase = sid * LINE_WORDS

    icp1 = pltpu.async_copy(l_hbm.at[pl.ds(base, BPW)], lidx, sem)
    icp2 = pltpu.async_copy(d_hbm.at[pl.ds(base, BPW)], didx, sem)
    icp1.wait()
    icp2.wait()

    def wave(q, carry):
        lvec = lidx[pl.ds(q * LANES, LANES)]
        dvec = didx[pl.ds(q * LANES, LANES)]
        for u in range(LANES):
            i = q * LANES + u
            lt = (lvec[u] // WIN) * WIN
            pltpu.async_copy(
                b4ld_hbm.at[pl.ds(dvec[u], 1), pl.ds(lt, WIN)].at[0],
                lines.at[pl.ds(sbase + i * WIN, WIN)], wsem)
        return carry

    lax.fori_loop(0, BPW // LANES, wave, 0)

    copies = []
    for k in range(NCH):
        isl = pl.ds(k * CHUNK, CHUNK)
        copies.append(pltpu.async_copy(b1_hbm.at[lidx.at[isl]], b1v.at[isl], sem))
        copies.append(pltpu.async_copy(b2_hbm.at[lidx.at[isl]], b2v.at[isl], sem))
        copies.append(pltpu.async_copy(b3_hbm.at[lidx.at[isl]], b3v.at[isl], sem))
        copies.append(pltpu.async_copy(b4_hbm.at[lidx.at[isl]], b4v.at[isl], sem))

    for k in range(NCH):
        for j in range(CHUNK // LANES):
            i0 = k * CHUNK + j * LANES
            sl = pl.ds(i0, LANES)
            lv = lidx[sl]
            fidx[k, pl.ds(j * LANES, LANES)] = (
                sbase + (lane_iota + i0) * WIN + (lv & (WIN - 1)))

    pltpu.make_async_copy(
        b1_hbm.at[pl.ds(0, LINE_WORDS)],
        lines.at[pl.ds(sbase, LINE_WORDS)], wsem).wait()

    ecopies = []
    for k in range(NCH):
        dsl = pl.ds(k * CHUNK, CHUNK)
        ecopies.append(pltpu.async_copy(
            lines.at[fidx.at[k]], b4e.at[dsl], wsem))
    for cp in ecopies:
        cp.wait()
    for cp in copies:
        cp.wait()

    for g in range(BPW // LANES):
        sl = pl.ds(g * LANES, LANES)
        b4v[sl] = b4v[sl] + b4e[sl]

    pltpu.sync_copy(b1v, b1_out.at[pl.ds(base, BPW)])
    pltpu.sync_copy(b2v, b2_out.at[pl.ds(base, BPW)])
    pltpu.sync_copy(b3v, b3_out.at[pl.ds(base, BPW)])
    pltpu.sync_copy(b4v, b4_out.at[pl.ds(base, BPW)])


def _tc_body(ct_ref, b1_ref, b2_ref, b3_ref, b4_ref, o_ref):
    b1 = b1_ref[...][None, :]
    b2 = b2_ref[...][None, :]
    b3 = b3_ref[...][None, :]
    b4 = b4_ref[...][None, :]
    ct = ct_ref[...]
    o_ref[...] = b1 + (b2 - b1) * jax.nn.sigmoid(b4 * (ct - b3))


_tc_sigmoid = pl.pallas_call(
    _tc_body,
    out_shape=jax.ShapeDtypeStruct((N_CONC, BATCH), jnp.float32),
    grid=(4,),
    in_specs=[
        pl.BlockSpec((N_CONC, BATCH // 4), lambda i: (0, i)),
        pl.BlockSpec((BATCH // 4,), lambda i: (i,)),
        pl.BlockSpec((BATCH // 4,), lambda i: (i,)),
        pl.BlockSpec((BATCH // 4,), lambda i: (i,)),
        pl.BlockSpec((BATCH // 4,), lambda i: (i,)),
    ],
    out_specs=pl.BlockSpec((N_CONC, BATCH // 4), lambda i: (0, i)),
)


def kernel(d, l, c, B1l, B2l, B3l, B4l, B4ld):
    b1, b2, b3, b4 = _sc_gather(d, l, B1l, B2l, B3l, B4l, B4ld.T)
    return _tc_sigmoid(c.T, b1, b2, b3, b4).T
